# baseline (speedup 1.0000x reference)
import functools

import jax
import jax.numpy as jnp
from jax import lax
from jax.experimental import pallas as pl
from jax.experimental.pallas import tpu as pltpu
from jax.experimental.pallas import tpu_sc as plsc

ALPHA = 8
FRAMES = 32


def _fast_body(x_ref, fast_ref):
    fast_ref[...] = x_ref[...]


_SC_MESH = plsc.VectorSubcoreMesh(core_axis_name="c", subcore_axis_name="s")


def _slow_body(x_hbm, slow_hbm, buf):
    wid = lax.axis_index("s") * 2 + lax.axis_index("c")

    @pl.when(wid < 24)
    def _():
        ch = wid // ALPHA
        s = wid % ALPHA
        pltpu.sync_copy(x_hbm.at[ch, 9 * s], buf)
        pltpu.sync_copy(buf, slow_hbm.at[ch, s])


def kernel(x):
    C, T, H, W = x.shape
    G = T // ALPHA
    fast = pl.pallas_call(
        _fast_body,
        grid=(C, T // FRAMES),
        in_specs=[pl.BlockSpec((1, FRAMES, H, W), lambda c, h: (c, h, 0, 0))],
        out_specs=pl.BlockSpec((1, FRAMES, H, W), lambda c, h: (c, h, 0, 0)),
        out_shape=jax.ShapeDtypeStruct((C, T, H, W), x.dtype),
    )(x)
    slow_fn = pl.kernel(
        _slow_body,
        out_type=jax.ShapeDtypeStruct((C, G, H, W), x.dtype),
        mesh=_SC_MESH,
        scratch_types=[pltpu.VMEM((H, W), x.dtype)],
    )
    slow = slow_fn(x)
    return (slow, fast)
